# baseline (speedup 1.0000x reference)
import functools

import jax
import jax.numpy as jnp
from jax import lax
from jax.experimental import pallas as pl
from jax.experimental.pallas import tpu as pltpu
from jax.experimental.pallas import tpu_sc as plsc

N = 10000
E = 320000
D = 128
HID = 128
NCLS = 2

NC = 2
NS = 16
NW = NC * NS
CHUNK = 128
SEG0, SEG1 = 80, 80
EG0, EG1 = 80, 80
DG0, DG1 = 80, 80
TOT_CHUNKS = NS * (SEG0 + SEG1)
E_PAD = TOT_CHUNKS * CHUNK
ROWS_PW = 632
N_PAD = NS * ROWS_PW
DEG_W = 128

_MESH = plsc.VectorSubcoreMesh(core_axis_name="c", subcore_axis_name="s",
                               num_cores=NC, num_subcores=NS)


IDXB = 8


def _my_chunks(c, s, cpw0, cpw1):
    chunk0 = jnp.where(c == 0, s * cpw0, NS * cpw0 + s * cpw1)
    nblk = jnp.where(c == 0, cpw0 // IDXB, cpw1 // IDXB)
    return chunk0, nblk


def _segsum_body(src_hbm, dst_hbm, h_hbm, z_hbm, csum_hbm,
                 srcv, dstv, buf0, buf1, cacc, sem0, sem1):
    c = lax.axis_index("c")
    s = lax.axis_index("s")
    chunk0, nblk = _my_chunks(c, s, SEG0, SEG1)
    rows = pl.ds(s * ROWS_PW, ROWS_PW)
    bufs = (buf0, buf1)
    sems = (sem0, sem1)
    pltpu.sync_copy(z_hbm.at[rows], cacc.at[rows])
    plsc.subcore_barrier()

    @pl.loop(0, nblk)
    def _(b):
        blk = pl.ds(chunk0 + b * IDXB, IDXB)
        pltpu.sync_copy(src_hbm.at[blk], srcv)
        pltpu.sync_copy(dst_hbm.at[blk], dstv)
        d = pltpu.async_copy(h_hbm.at[srcv.at[0]], bufs[0], sems[0])
        for jj in range(IDXB):
            if jj + 1 < IDXB:
                d_next = pltpu.async_copy(
                    h_hbm.at[srcv.at[jj + 1]], bufs[(jj + 1) % 2],
                    sems[(jj + 1) % 2])
            d.wait()
            pltpu.sync_copy(bufs[jj % 2], cacc.at[dstv.at[jj]], add=True)
            if jj + 1 < IDXB:
                d = d_next

    plsc.subcore_barrier()
    pltpu.sync_copy(cacc.at[rows], csum_hbm.at[c, rows])


def _segsum(src2, dst2, h):
    z = jnp.zeros((N_PAD, D), jnp.float32)
    fn = pl.kernel(
        _segsum_body,
        out_type=jax.ShapeDtypeStruct((NC, N_PAD, D), jnp.float32),
        mesh=_MESH,
        scratch_types=[
            pltpu.VMEM((IDXB, CHUNK), jnp.int32),
            pltpu.VMEM((IDXB, CHUNK), jnp.int32),
            pltpu.VMEM((CHUNK, D), jnp.float32),
            pltpu.VMEM((CHUNK, D), jnp.float32),
            pltpu.VMEM_SHARED((N_PAD, D), jnp.float32),
            pltpu.SemaphoreType.DMA,
            pltpu.SemaphoreType.DMA,
        ],
    )
    return fn(src2, dst2, h, z)


def _degree_body(dst_hbm, z16_hbm, ones_hbm, deg_hbm, dstv, onesv, dacc):
    c = lax.axis_index("c")
    s = lax.axis_index("s")
    chunk0, nblk = _my_chunks(c, s, DG0, DG1)
    rows = pl.ds(s * ROWS_PW, ROWS_PW)
    pltpu.sync_copy(z16_hbm.at[rows], dacc.at[rows])
    pltpu.sync_copy(ones_hbm, onesv)
    plsc.subcore_barrier()

    @pl.loop(0, nblk)
    def _(b):
        pltpu.sync_copy(dst_hbm.at[pl.ds(chunk0 + b * IDXB, IDXB)], dstv)
        for jj in range(IDXB):
            pltpu.sync_copy(onesv, dacc.at[dstv.at[jj]], add=True)

    plsc.subcore_barrier()
    pltpu.sync_copy(dacc.at[rows], deg_hbm.at[c, rows])


def _degree(dst2):
    fn = pl.kernel(
        _degree_body,
        out_type=jax.ShapeDtypeStruct((NC, N_PAD, DEG_W), jnp.float32),
        mesh=_MESH,
        scratch_types=[
            pltpu.VMEM((IDXB, CHUNK), jnp.int32),
            pltpu.VMEM((CHUNK, DEG_W), jnp.float32),
            pltpu.VMEM_SHARED((N_PAD, DEG_W), jnp.float32),
        ],
    )
    return fn(dst2, jnp.zeros((N_PAD, DEG_W), jnp.float32),
              jnp.ones((CHUNK, DEG_W), jnp.float32))


def _edge_gather_body(src_hbm, dst_hbm, a_hbm, b_hbm, s_hbm,
                      srcv, dstv, bufa0, bufa1, bufb0, bufb1,
                      sema0, sema1, semb0, semb1):
    c = lax.axis_index("c")
    s = lax.axis_index("s")
    chunk0, nblk = _my_chunks(c, s, EG0, EG1)
    bufas = (bufa0, bufa1)
    bufbs = (bufb0, bufb1)
    semas = (sema0, sema1)
    sembs = (semb0, semb1)

    @pl.loop(0, nblk)
    def _(b):
        blk0 = chunk0 + b * IDXB
        pltpu.sync_copy(src_hbm.at[pl.ds(blk0, IDXB)], srcv)
        pltpu.sync_copy(dst_hbm.at[pl.ds(blk0, IDXB)], dstv)
        da = pltpu.async_copy(a_hbm.at[srcv.at[0]], bufas[0], semas[0])
        db = pltpu.async_copy(b_hbm.at[dstv.at[0]], bufbs[0], sembs[0])
        for jj in range(IDXB):
            if jj + 1 < IDXB:
                k = (jj + 1) % 2
                da_n = pltpu.async_copy(a_hbm.at[srcv.at[jj + 1]],
                                        bufas[k], semas[k])
                db_n = pltpu.async_copy(b_hbm.at[dstv.at[jj + 1]],
                                        bufbs[k], sembs[k])
            da.wait()
            db.wait()
            ba = bufas[jj % 2]
            bb = bufbs[jj % 2]

            @pl.loop(0, CHUNK)
            def _(r):
                for cs_ in range(D // 16):
                    sl = pl.ds(cs_ * 16, 16)
                    ba[r, sl] = ba[r, sl] + bb[r, sl]

            pltpu.sync_copy(ba, s_hbm.at[pl.ds((blk0 + jj) * CHUNK, CHUNK)])
            if jj + 1 < IDXB:
                da, db = da_n, db_n


def _edge_gather(src2, dst2, a, b):
    fn = pl.kernel(
        _edge_gather_body,
        out_type=jax.ShapeDtypeStruct((E_PAD, D), jnp.float32),
        mesh=_MESH,
        scratch_types=[
            pltpu.VMEM((IDXB, CHUNK), jnp.int32),
            pltpu.VMEM((IDXB, CHUNK), jnp.int32),
            pltpu.VMEM((CHUNK, D), jnp.float32),
            pltpu.VMEM((CHUNK, D), jnp.float32),
            pltpu.VMEM((CHUNK, D), jnp.float32),
            pltpu.VMEM((CHUNK, D), jnp.float32),
            pltpu.SemaphoreType.DMA,
            pltpu.SemaphoreType.DMA,
            pltpu.SemaphoreType.DMA,
            pltpu.SemaphoreType.DMA,
        ],
    )
    return fn(src2, dst2, a, b)


BLK_N = 1000
BLK_E = 2000


def _embed_tc(h, w, b):
    def body(h_ref, w_ref, b_ref, o_ref):
        o_ref[...] = (jnp.dot(h_ref[...], w_ref[...],
                              preferred_element_type=jnp.float32)
                      + b_ref[...])

    return pl.pallas_call(
        body,
        grid=(N // BLK_N,),
        in_specs=[pl.BlockSpec((BLK_N, D), lambda i: (i, 0)),
                  pl.BlockSpec((D, HID), lambda i: (0, 0)),
                  pl.BlockSpec((1, HID), lambda i: (0, 0))],
        out_specs=pl.BlockSpec((BLK_N, HID), lambda i: (i, 0)),
        out_shape=jax.ShapeDtypeStruct((N, HID), jnp.float32),
    )(h, w, b.reshape(1, HID))


def _layer_tc(h, csum, deg, wt, wb, b):
    def body(h_ref, c_ref, d_ref, wt_ref, wb_ref, b_ref, o_ref):
        hblk = h_ref[...]
        cs = c_ref[0] + c_ref[1]
        dg = d_ref[0, :, 0:1] + d_ref[1, :, 0:1]
        cs = cs / jnp.maximum(dg, 1.0)
        t = (jnp.dot(hblk, wt_ref[...], preferred_element_type=jnp.float32)
             + jnp.dot(cs, wb_ref[...], preferred_element_type=jnp.float32)
             + b_ref[...])
        nrm = jnp.sqrt(jnp.sum(t * t, axis=1, keepdims=True))
        t = t / jnp.maximum(nrm, 1e-12)
        o_ref[...] = hblk + jnp.maximum(t, 0.0)

    return pl.pallas_call(
        body,
        grid=(N // BLK_N,),
        in_specs=[pl.BlockSpec((BLK_N, HID), lambda i: (i, 0)),
                  pl.BlockSpec((NC, BLK_N, HID), lambda i: (0, i, 0)),
                  pl.BlockSpec((NC, BLK_N, DEG_W), lambda i: (0, i, 0)),
                  pl.BlockSpec((HID, HID), lambda i: (0, 0)),
                  pl.BlockSpec((HID, HID), lambda i: (0, 0)),
                  pl.BlockSpec((1, HID), lambda i: (0, 0))],
        out_specs=pl.BlockSpec((BLK_N, HID), lambda i: (i, 0)),
        out_shape=jax.ShapeDtypeStruct((N, HID), jnp.float32),
    )(h, csum, deg, wt, wb, b.reshape(1, HID))


def _edge_pre_tc(h, w0t, w0b, b0):
    def body(h_ref, wt_ref, wb_ref, b_ref, a_ref, bm_ref):
        hblk = h_ref[...]
        a_ref[...] = (jnp.dot(hblk, wt_ref[...],
                              preferred_element_type=jnp.float32)
                      + b_ref[...])
        bm_ref[...] = jnp.dot(hblk, wb_ref[...],
                              preferred_element_type=jnp.float32)

    return pl.pallas_call(
        body,
        grid=(N // BLK_N,),
        in_specs=[pl.BlockSpec((BLK_N, HID), lambda i: (i, 0)),
                  pl.BlockSpec((HID, HID), lambda i: (0, 0)),
                  pl.BlockSpec((HID, HID), lambda i: (0, 0)),
                  pl.BlockSpec((1, HID), lambda i: (0, 0))],
        out_specs=[pl.BlockSpec((BLK_N, HID), lambda i: (i, 0)),
                   pl.BlockSpec((BLK_N, HID), lambda i: (i, 0))],
        out_shape=[jax.ShapeDtypeStruct((N, HID), jnp.float32),
                   jax.ShapeDtypeStruct((N, HID), jnp.float32)],
    )(h, w0t, w0b, b0.reshape(1, HID))


def _final_tc(sv, w1, b1, w2, b2):
    def body(s_ref, w1_ref, b1_ref, w2_ref, b2_ref, o_ref):
        y = jnp.maximum(s_ref[...], 0.0).astype(jnp.bfloat16)
        y = jnp.maximum(jnp.dot(y, w1_ref[...].astype(jnp.bfloat16),
                                preferred_element_type=jnp.float32)
                        + b1_ref[...], 0.0).astype(jnp.bfloat16)
        o_ref[...] = (jnp.dot(y, w2_ref[...].astype(jnp.bfloat16),
                              preferred_element_type=jnp.float32)
                      + b2_ref[...])

    return pl.pallas_call(
        body,
        grid=(E // BLK_E,),
        in_specs=[pl.BlockSpec((BLK_E, HID), lambda i: (i, 0)),
                  pl.BlockSpec((HID, HID // 2), lambda i: (0, 0)),
                  pl.BlockSpec((1, HID // 2), lambda i: (0, 0)),
                  pl.BlockSpec((HID // 2, NCLS), lambda i: (0, 0)),
                  pl.BlockSpec((1, NCLS), lambda i: (0, 0))],
        out_specs=pl.BlockSpec((BLK_E, NCLS), lambda i: (i, 0)),
        out_shape=jax.ShapeDtypeStruct((E, NCLS), jnp.float32),
    )(sv, w1, b1.reshape(1, HID // 2), w2, b2.reshape(1, NCLS))


def kernel(edge_index, h, e, snorm_n, snorm_e, W_emb, b_emb,
           W_l0, b_l0, W_l1, b_l1, W_l2, b_l2, W_l3, b_l3,
           W_mlp0, b_mlp0, W_mlp1, b_mlp1, W_mlp2, b_mlp2):
    src = edge_index[0].astype(jnp.int32)
    dst = edge_index[1].astype(jnp.int32)
    pad = E_PAD - E
    pad_ix = jnp.arange(pad, dtype=jnp.int32)
    pad_src = (pad_ix * 97) % N
    pad_dst = N + pad_ix % (N_PAD - N)
    src2 = jnp.concatenate([src, pad_src]).reshape(TOT_CHUNKS, CHUNK)
    dst2 = jnp.concatenate([dst, pad_dst]).reshape(TOT_CHUNKS, CHUNK)

    hx = _embed_tc(h.astype(jnp.float32), W_emb, b_emb)
    deg = _degree(dst2)

    for w, b in ((W_l0, b_l0), (W_l1, b_l1), (W_l2, b_l2), (W_l3, b_l3)):
        csum = _segsum(src2, dst2, hx)
        hx = _layer_tc(hx, csum, deg, w[:HID], w[HID:], b)

    a, bm = _edge_pre_tc(hx, W_mlp0[:HID], W_mlp0[HID:], b_mlp0)
    sv = _edge_gather(src2, dst2, a, bm)
    return _final_tc(sv, W_mlp1, b_mlp1, W_mlp2, b_mlp2)


# speedup vs baseline: 1.0145x; 1.0145x over previous
import functools

import jax
import jax.numpy as jnp
from jax import lax
from jax.experimental import pallas as pl
from jax.experimental.pallas import tpu as pltpu
from jax.experimental.pallas import tpu_sc as plsc

N = 10000
E = 320000
D = 128
HID = 128
NCLS = 2

NC = 2
NS = 16
NW = NC * NS
CHUNK = 128
SEG0, SEG1 = 80, 80
EG0, EG1 = 80, 80
DG0, DG1 = 80, 80
TOT_CHUNKS = NS * (SEG0 + SEG1)
E_PAD = TOT_CHUNKS * CHUNK
ROWS_PW = 632
N_PAD = NS * ROWS_PW
DEG_W = 128

_MESH = plsc.VectorSubcoreMesh(core_axis_name="c", subcore_axis_name="s",
                               num_cores=NC, num_subcores=NS)


IDXB = 8


def _my_chunks(c, s, cpw0, cpw1):
    chunk0 = jnp.where(c == 0, s * cpw0, NS * cpw0 + s * cpw1)
    nblk = jnp.where(c == 0, cpw0 // IDXB, cpw1 // IDXB)
    return chunk0, nblk


def _segsum_body(src_hbm, dst_hbm, h_hbm, z_hbm, csum_hbm,
                 srcv, dstv, buf0, buf1, cacc, sem0, sem1):
    c = lax.axis_index("c")
    s = lax.axis_index("s")
    chunk0, nblk = _my_chunks(c, s, SEG0, SEG1)
    rows = pl.ds(s * ROWS_PW, ROWS_PW)
    bufs = (buf0, buf1)
    sems = (sem0, sem1)
    pltpu.sync_copy(z_hbm.at[rows], cacc.at[rows])
    plsc.subcore_barrier()

    @pl.loop(0, nblk)
    def _(b):
        blk = pl.ds(chunk0 + b * IDXB, IDXB)
        pltpu.sync_copy(src_hbm.at[blk], srcv)
        pltpu.sync_copy(dst_hbm.at[blk], dstv)
        d = pltpu.async_copy(h_hbm.at[srcv.at[0]], bufs[0], sems[0])
        for jj in range(IDXB):
            if jj + 1 < IDXB:
                d_next = pltpu.async_copy(
                    h_hbm.at[srcv.at[jj + 1]], bufs[(jj + 1) % 2],
                    sems[(jj + 1) % 2])
            d.wait()
            pltpu.sync_copy(bufs[jj % 2], cacc.at[dstv.at[jj]], add=True)
            if jj + 1 < IDXB:
                d = d_next

    plsc.subcore_barrier()
    pltpu.sync_copy(cacc.at[rows], csum_hbm.at[c, rows])


def _segsum(src2, dst2, h):
    z = jnp.zeros((N_PAD, D), jnp.float32)
    fn = pl.kernel(
        _segsum_body,
        out_type=jax.ShapeDtypeStruct((NC, N_PAD, D), jnp.float32),
        mesh=_MESH,
        scratch_types=[
            pltpu.VMEM((IDXB, CHUNK), jnp.int32),
            pltpu.VMEM((IDXB, CHUNK), jnp.int32),
            pltpu.VMEM((CHUNK, D), jnp.float32),
            pltpu.VMEM((CHUNK, D), jnp.float32),
            pltpu.VMEM_SHARED((N_PAD, D), jnp.float32),
            pltpu.SemaphoreType.DMA,
            pltpu.SemaphoreType.DMA,
        ],
    )
    return fn(src2, dst2, h, z)


def _degree_body(dst_hbm, z16_hbm, ones_hbm, deg_hbm, dstv, onesv, dacc):
    c = lax.axis_index("c")
    s = lax.axis_index("s")
    chunk0, nblk = _my_chunks(c, s, DG0, DG1)
    rows = pl.ds(s * ROWS_PW, ROWS_PW)
    pltpu.sync_copy(z16_hbm.at[rows], dacc.at[rows])
    pltpu.sync_copy(ones_hbm, onesv)
    plsc.subcore_barrier()

    @pl.loop(0, nblk)
    def _(b):
        pltpu.sync_copy(dst_hbm.at[pl.ds(chunk0 + b * IDXB, IDXB)], dstv)
        for jj in range(IDXB):
            pltpu.sync_copy(onesv, dacc.at[dstv.at[jj]], add=True)

    plsc.subcore_barrier()
    pltpu.sync_copy(dacc.at[rows], deg_hbm.at[c, rows])


def _degree(dst2):
    fn = pl.kernel(
        _degree_body,
        out_type=jax.ShapeDtypeStruct((NC, N_PAD, DEG_W), jnp.float32),
        mesh=_MESH,
        scratch_types=[
            pltpu.VMEM((IDXB, CHUNK), jnp.int32),
            pltpu.VMEM((CHUNK, DEG_W), jnp.float32),
            pltpu.VMEM_SHARED((N_PAD, DEG_W), jnp.float32),
        ],
    )
    return fn(dst2, jnp.zeros((N_PAD, DEG_W), jnp.float32),
              jnp.ones((CHUNK, DEG_W), jnp.float32))


def _edge_gather_body(src_hbm, dst_hbm, a_hbm, b_hbm, s_hbm,
                      srcv, dstv, bufa0, bufa1, bufb0, bufb1,
                      sema0, sema1, semb0, semb1):
    c = lax.axis_index("c")
    s = lax.axis_index("s")
    chunk0, nblk = _my_chunks(c, s, EG0, EG1)
    bufas = (bufa0, bufa1)
    bufbs = (bufb0, bufb1)
    semas = (sema0, sema1)
    sembs = (semb0, semb1)

    @pl.loop(0, nblk)
    def _(b):
        blk0 = chunk0 + b * IDXB
        pltpu.sync_copy(src_hbm.at[pl.ds(blk0, IDXB)], srcv)
        pltpu.sync_copy(dst_hbm.at[pl.ds(blk0, IDXB)], dstv)
        da = pltpu.async_copy(a_hbm.at[srcv.at[0]], bufas[0], semas[0])
        db = pltpu.async_copy(b_hbm.at[dstv.at[0]], bufbs[0], sembs[0])
        for jj in range(IDXB):
            if jj + 1 < IDXB:
                k = (jj + 1) % 2
                da_n = pltpu.async_copy(a_hbm.at[srcv.at[jj + 1]],
                                        bufas[k], semas[k])
                db_n = pltpu.async_copy(b_hbm.at[dstv.at[jj + 1]],
                                        bufbs[k], sembs[k])
            da.wait()
            db.wait()
            ba = bufas[jj % 2]
            bb = bufbs[jj % 2]

            @pl.loop(0, CHUNK)
            def _(r):
                for cs_ in range(D // 16):
                    sl = pl.ds(cs_ * 16, 16)
                    ba[r, sl] = ba[r, sl] + bb[r, sl]

            pltpu.sync_copy(ba, s_hbm.at[pl.ds((blk0 + jj) * CHUNK, CHUNK)])
            if jj + 1 < IDXB:
                da, db = da_n, db_n


def _edge_gather(src2, dst2, a, b):
    fn = pl.kernel(
        _edge_gather_body,
        out_type=jax.ShapeDtypeStruct((E_PAD, D), jnp.float32),
        mesh=_MESH,
        scratch_types=[
            pltpu.VMEM((IDXB, CHUNK), jnp.int32),
            pltpu.VMEM((IDXB, CHUNK), jnp.int32),
            pltpu.VMEM((CHUNK, D), jnp.float32),
            pltpu.VMEM((CHUNK, D), jnp.float32),
            pltpu.VMEM((CHUNK, D), jnp.float32),
            pltpu.VMEM((CHUNK, D), jnp.float32),
            pltpu.SemaphoreType.DMA,
            pltpu.SemaphoreType.DMA,
            pltpu.SemaphoreType.DMA,
            pltpu.SemaphoreType.DMA,
        ],
    )
    return fn(src2, dst2, a, b)


BLK_N = 1000
BLK_E = 2000


def _embed_tc(h, w, b):
    def body(h_ref, w_ref, b_ref, o_ref):
        o_ref[...] = (jnp.dot(h_ref[...], w_ref[...],
                              preferred_element_type=jnp.float32)
                      + b_ref[...])

    return pl.pallas_call(
        body,
        grid=(N // BLK_N,),
        in_specs=[pl.BlockSpec((BLK_N, D), lambda i: (i, 0)),
                  pl.BlockSpec((D, HID), lambda i: (0, 0)),
                  pl.BlockSpec((1, HID), lambda i: (0, 0))],
        out_specs=pl.BlockSpec((BLK_N, HID), lambda i: (i, 0)),
        out_shape=jax.ShapeDtypeStruct((N, HID), jnp.float32),
    )(h, w, b.reshape(1, HID))


def _layer_tc(h, csum, deg, wt, wb, b):
    def body(h_ref, c_ref, d_ref, wt_ref, wb_ref, b_ref, o_ref):
        hblk = h_ref[...]
        cs = c_ref[0] + c_ref[1]
        dg = d_ref[0, :, 0:1] + d_ref[1, :, 0:1]
        cs = cs / jnp.maximum(dg, 1.0)
        t = (jnp.dot(hblk, wt_ref[...], preferred_element_type=jnp.float32)
             + jnp.dot(cs, wb_ref[...], preferred_element_type=jnp.float32)
             + b_ref[...])
        nrm = jnp.sqrt(jnp.sum(t * t, axis=1, keepdims=True))
        t = t / jnp.maximum(nrm, 1e-12)
        o_ref[...] = hblk + jnp.maximum(t, 0.0)

    return pl.pallas_call(
        body,
        grid=(N // BLK_N,),
        in_specs=[pl.BlockSpec((BLK_N, HID), lambda i: (i, 0)),
                  pl.BlockSpec((NC, BLK_N, HID), lambda i: (0, i, 0)),
                  pl.BlockSpec((NC, BLK_N, DEG_W), lambda i: (0, i, 0)),
                  pl.BlockSpec((HID, HID), lambda i: (0, 0)),
                  pl.BlockSpec((HID, HID), lambda i: (0, 0)),
                  pl.BlockSpec((1, HID), lambda i: (0, 0))],
        out_specs=pl.BlockSpec((BLK_N, HID), lambda i: (i, 0)),
        out_shape=jax.ShapeDtypeStruct((N, HID), jnp.float32),
    )(h, csum, deg, wt, wb, b.reshape(1, HID))


def _edge_pre_tc(h, w0t, w0b, b0):
    def body(h_ref, wt_ref, wb_ref, b_ref, a_ref, bm_ref):
        hblk = h_ref[...]
        a_ref[...] = (jnp.dot(hblk, wt_ref[...],
                              preferred_element_type=jnp.float32)
                      + b_ref[...])
        bm_ref[...] = jnp.dot(hblk, wb_ref[...],
                              preferred_element_type=jnp.float32)

    return pl.pallas_call(
        body,
        grid=(N // BLK_N,),
        in_specs=[pl.BlockSpec((BLK_N, HID), lambda i: (i, 0)),
                  pl.BlockSpec((HID, HID), lambda i: (0, 0)),
                  pl.BlockSpec((HID, HID), lambda i: (0, 0)),
                  pl.BlockSpec((1, HID), lambda i: (0, 0))],
        out_specs=[pl.BlockSpec((BLK_N, HID), lambda i: (i, 0)),
                   pl.BlockSpec((BLK_N, HID), lambda i: (i, 0))],
        out_shape=[jax.ShapeDtypeStruct((N, HID), jnp.float32),
                   jax.ShapeDtypeStruct((N, HID), jnp.float32)],
    )(h, w0t, w0b, b0.reshape(1, HID))


def _final_tc(sv, w1, b1, w2, b2):
    def body(s_ref, w1_ref, b1_ref, w2_ref, b2_ref, o_ref):
        y = jnp.maximum(s_ref[...], 0.0)
        y = jnp.maximum(jnp.dot(y, w1_ref[...],
                                preferred_element_type=jnp.float32)
                        + b1_ref[...], 0.0)
        o_ref[...] = (jnp.dot(y, w2_ref[...],
                              preferred_element_type=jnp.float32)
                      + b2_ref[...])

    return pl.pallas_call(
        body,
        grid=(E // BLK_E,),
        in_specs=[pl.BlockSpec((BLK_E, HID), lambda i: (i, 0)),
                  pl.BlockSpec((HID, HID // 2), lambda i: (0, 0)),
                  pl.BlockSpec((1, HID // 2), lambda i: (0, 0)),
                  pl.BlockSpec((HID // 2, NCLS), lambda i: (0, 0)),
                  pl.BlockSpec((1, NCLS), lambda i: (0, 0))],
        out_specs=pl.BlockSpec((BLK_E, NCLS), lambda i: (i, 0)),
        out_shape=jax.ShapeDtypeStruct((E, NCLS), jnp.float32),
    )(sv, w1, b1.reshape(1, HID // 2), w2, b2.reshape(1, NCLS))


def kernel(edge_index, h, e, snorm_n, snorm_e, W_emb, b_emb,
           W_l0, b_l0, W_l1, b_l1, W_l2, b_l2, W_l3, b_l3,
           W_mlp0, b_mlp0, W_mlp1, b_mlp1, W_mlp2, b_mlp2):
    src = edge_index[0].astype(jnp.int32)
    dst = edge_index[1].astype(jnp.int32)
    pad = E_PAD - E
    pad_ix = jnp.arange(pad, dtype=jnp.int32)
    pad_src = (pad_ix * 97) % N
    pad_dst = N + pad_ix % (N_PAD - N)
    src2 = jnp.concatenate([src, pad_src]).reshape(TOT_CHUNKS, CHUNK)
    dst2 = jnp.concatenate([dst, pad_dst]).reshape(TOT_CHUNKS, CHUNK)

    hx = _embed_tc(h.astype(jnp.float32), W_emb, b_emb)
    deg = _degree(dst2)

    for w, b in ((W_l0, b_l0), (W_l1, b_l1), (W_l2, b_l2), (W_l3, b_l3)):
        csum = _segsum(src2, dst2, hx)
        hx = _layer_tc(hx, csum, deg, w[:HID], w[HID:], b)

    a, bm = _edge_pre_tc(hx, W_mlp0[:HID], W_mlp0[HID:], b_mlp0)
    sv = _edge_gather(src2, dst2, a, bm)
    return _final_tc(sv, W_mlp1, b_mlp1, W_mlp2, b_mlp2)
